# Initial kernel scaffold; baseline (speedup 1.0000x reference)
import jax
import jax.numpy as jnp
from jax.experimental import pallas as pl


def kernel(x_nchw, w1, b1, w2, b2):
    raise NotImplementedError("write your pallas kernel here")


# speedup vs baseline: 1.4697x; 1.4697x over previous
import functools

import jax
import jax.numpy as jnp
from jax.experimental import pallas as pl
from jax.experimental.pallas import tpu as pltpu

_F32 = jnp.float32


def _se_fused_kernel(x_ref, w1_ref, b1_ref, w2_ref, b2_ref, o_ref, *, inv_hw):
    xs = x_ref[0].astype(_F32)
    mean = jnp.sum(xs, axis=1, keepdims=True) * inv_hw
    h = jnp.dot(w1_ref[...], mean, preferred_element_type=_F32)
    h = jnp.maximum(h + b1_ref[...], 0.0)
    s = jnp.dot(w2_ref[...], h, preferred_element_type=_F32)
    scale = jax.nn.sigmoid(s + b2_ref[...])
    o_ref[0] = (xs * scale).astype(o_ref.dtype)


def kernel(x_nchw, w1, b1, w2, b2):
    N, C, H, W = x_nchw.shape
    HW = H * W
    mid = w1.shape[0]

    x3 = x_nchw.reshape(N, C, HW)
    w1m = w1.reshape(mid, C).astype(_F32)
    b1m = b1.reshape(mid, 1).astype(_F32)
    w2m = w2.reshape(C, mid).astype(_F32)
    b2m = b2.reshape(C, 1).astype(_F32)

    out3 = pl.pallas_call(
        functools.partial(_se_fused_kernel, inv_hw=1.0 / HW),
        out_shape=jax.ShapeDtypeStruct((N, C, HW), x_nchw.dtype),
        grid=(N,),
        in_specs=[
            pl.BlockSpec((1, C, HW), lambda n: (n, 0, 0)),
            pl.BlockSpec((mid, C), lambda n: (0, 0)),
            pl.BlockSpec((mid, 1), lambda n: (0, 0)),
            pl.BlockSpec((C, mid), lambda n: (0, 0)),
            pl.BlockSpec((C, 1), lambda n: (0, 0)),
        ],
        out_specs=pl.BlockSpec((1, C, HW), lambda n: (n, 0, 0)),
        compiler_params=pltpu.CompilerParams(
            dimension_semantics=("parallel",)),
    )(x3, w1m, b1m, w2m, b2m)

    return out3.reshape(N, C, H, W)
